# baseline (speedup 1.0000x reference)
import jax
import jax.numpy as jnp
from jax import lax
from jax.experimental import pallas as pl
from jax.experimental.pallas import tpu as pltpu
from jax.experimental.pallas import tpu_sc as plsc

D = 64
LANE = 128
NC, NS = 2, 16
NW = NC * NS
NB = 2
K = 2
VL = 16

VOCAB = 1000000
MAIN_BLKS = VOCAB // LANE
TAIL = VOCAB - MAIN_BLKS * LANE
BASE_BLKS = MAIN_BLKS // NW
EXTRA = MAIN_BLKS - BASE_BLKS * NW


def _fmt_body(wt_hbm, wtail_hbm, w128_hbm, v64, v128, vtail):
    c = lax.axis_index("c")
    s = lax.axis_index("s")
    wid = s * NC + c
    nblk = BASE_BLKS + jnp.where(wid < EXTRA, 1, 0)
    start = wid * BASE_BLKS + jnp.minimum(wid, EXTRA)

    row_idx = [jnp.arange(cc * VL, (cc + 1) * VL, dtype=jnp.int32)
               for cc in range(D // VL)]

    def transpose_into(src, dst, ncols):
        for i in range(ncols):
            col = jnp.full((VL,), i, dtype=jnp.int32)
            for cc in range(D // VL):
                x = plsc.load_gather(src, [row_idx[cc], col])
                dst[i, pl.ds(cc * VL, VL)] = x

    def do_block(i, carry):
        blk = start + i
        r0 = blk * LANE
        pltpu.sync_copy(wt_hbm.at[:, pl.ds(r0, LANE)], v64)
        transpose_into(v64, v128, LANE)
        pltpu.sync_copy(v128, w128_hbm.at[pl.ds(r0, LANE)])
        return carry

    lax.fori_loop(0, nblk, do_block, 0)

    @pl.when(wid == 0)
    def _():
        pltpu.sync_copy(wtail_hbm, vtail)
        transpose_into(vtail, v128, TAIL)
        pltpu.sync_copy(v128.at[pl.ds(0, TAIL)],
                        w128_hbm.at[pl.ds(MAIN_BLKS * LANE, TAIL)])


def _emb_body(idx_hbm, w_hbm, out_hbm, idx_v, rows_v, gsem):
    tok_rows = idx_hbm.shape[0]
    rpw = tok_rows // NW
    ch = rpw // K

    c = lax.axis_index("c")
    s = lax.axis_index("s")
    wid = s * NC + c
    base = wid * rpw

    def start(g, b):
        r0 = base + g * K
        pltpu.sync_copy(idx_hbm.at[pl.ds(r0, K)], idx_v.at[b])
        for j in range(K):
            pltpu.async_copy(w_hbm.at[idx_v.at[b, j]], rows_v.at[b, j],
                             gsem.at[b])

    def finish(g, b):
        r0 = base + g * K
        for j in range(K):
            pltpu.make_async_copy(w_hbm.at[pl.ds(0, LANE)], rows_v.at[b, j],
                                  gsem.at[b]).wait()
        pltpu.sync_copy(rows_v.at[b], out_hbm.at[pl.ds(r0, K)])

    start(0, 0)
    start(1, 1)

    def loop_body(i, carry):
        g = 2 * i
        finish(g, 0)
        start(g + 2, 0)
        finish(g + 1, 1)
        start(g + 3, 1)
        return carry

    lax.fori_loop(0, (ch - 2) // 2, loop_body, 0)
    finish(ch - 2, 0)
    finish(ch - 1, 1)


def kernel(tokens, weight):
    b0, b1 = tokens.shape
    tok_rows = (b0 * b1) // LANE
    idx = tokens.reshape(tok_rows, LANE)
    wt = weight.T
    wt_main = wt
    wt_tail = wt[:, MAIN_BLKS * LANE:]
    mesh = plsc.VectorSubcoreMesh(core_axis_name="c", subcore_axis_name="s",
                                  num_cores=NC, num_subcores=NS)
    w128 = pl.kernel(
        _fmt_body,
        out_type=jax.ShapeDtypeStruct((VOCAB, LANE), jnp.float32),
        mesh=mesh,
        scratch_types=[
            pltpu.VMEM((D, LANE), jnp.float32),
            pltpu.VMEM((LANE, LANE), jnp.float32),
            pltpu.VMEM((D, TAIL), jnp.float32),
        ],
        compiler_params=pltpu.CompilerParams(needs_layout_passes=False),
    )(wt_main, wt_tail)
    out = pl.kernel(
        _emb_body,
        out_type=jax.ShapeDtypeStruct((tok_rows, LANE, LANE), jnp.float32),
        mesh=mesh,
        scratch_types=[
            pltpu.VMEM((NB, K, LANE), jnp.int32),
            pltpu.VMEM((NB, K, LANE, LANE), jnp.float32),
            pltpu.SemaphoreType.DMA((NB,)),
        ],
    )(idx, w128)
    return out[:, :, :D].reshape(b0, b1, D)


# speedup vs baseline: 1.4429x; 1.4429x over previous
import jax
import jax.numpy as jnp
from jax import lax
from jax.experimental import pallas as pl
from jax.experimental.pallas import tpu as pltpu
from jax.experimental.pallas import tpu_sc as plsc

D = 64
LANE = 128
NC, NS = 2, 16
NW = NC * NS
NB = 2
K = 2
VL = 16

VOCAB = 1000000
MAIN_BLKS = VOCAB // LANE
TAIL = VOCAB - MAIN_BLKS * LANE
BASE_BLKS = MAIN_BLKS // NW
EXTRA = MAIN_BLKS - BASE_BLKS * NW


def _fmt_body(wt_hbm, wtail_hbm, w128_hbm, v64, v128, vtail):
    c = lax.axis_index("c")
    s = lax.axis_index("s")
    wid = s * NC + c
    nblk = BASE_BLKS + jnp.where(wid < EXTRA, 1, 0)
    start = wid * BASE_BLKS + jnp.minimum(wid, EXTRA)

    row_idx = [jnp.arange(cc * VL, (cc + 1) * VL, dtype=jnp.int32)
               for cc in range(D // VL)]

    def transpose_into(src, dst, ncols):
        @plsc.parallel_loop(0, ncols, unroll=8)
        def _(i):
            col = jnp.full((VL,), i, dtype=jnp.int32)
            for cc in range(D // VL):
                x = plsc.load_gather(src, [row_idx[cc], col])
                dst[i, pl.ds(cc * VL, VL)] = x

    def do_block(i, carry):
        blk = start + i
        r0 = blk * LANE
        pltpu.sync_copy(wt_hbm.at[:, pl.ds(r0, LANE)], v64)
        transpose_into(v64, v128, LANE)
        pltpu.sync_copy(v128, w128_hbm.at[pl.ds(r0, LANE)])
        return carry

    lax.fori_loop(0, nblk, do_block, 0)

    @pl.when(wid == 0)
    def _():
        pltpu.sync_copy(wtail_hbm, vtail)
        transpose_into(vtail, v128, TAIL)
        pltpu.sync_copy(v128.at[pl.ds(0, TAIL)],
                        w128_hbm.at[pl.ds(MAIN_BLKS * LANE, TAIL)])


def _emb_body(idx_hbm, w_hbm, out_hbm, idx_v, rows_v, gsem):
    tok_rows = idx_hbm.shape[0]
    rpw = tok_rows // NW
    ch = rpw // K

    c = lax.axis_index("c")
    s = lax.axis_index("s")
    wid = s * NC + c
    base = wid * rpw

    def start(g, b):
        r0 = base + g * K
        pltpu.sync_copy(idx_hbm.at[pl.ds(r0, K)], idx_v.at[b])
        for j in range(K):
            pltpu.async_copy(w_hbm.at[idx_v.at[b, j]], rows_v.at[b, j],
                             gsem.at[b])

    def finish(g, b):
        r0 = base + g * K
        for j in range(K):
            pltpu.make_async_copy(w_hbm.at[pl.ds(0, LANE)], rows_v.at[b, j],
                                  gsem.at[b]).wait()
        pltpu.sync_copy(rows_v.at[b], out_hbm.at[pl.ds(r0, K)])

    start(0, 0)
    start(1, 1)

    def loop_body(i, carry):
        g = 2 * i
        finish(g, 0)
        start(g + 2, 0)
        finish(g + 1, 1)
        start(g + 3, 1)
        return carry

    lax.fori_loop(0, (ch - 2) // 2, loop_body, 0)
    finish(ch - 2, 0)
    finish(ch - 1, 1)


def kernel(tokens, weight):
    b0, b1 = tokens.shape
    tok_rows = (b0 * b1) // LANE
    idx = tokens.reshape(tok_rows, LANE)
    wt = weight.T
    wt_main = wt
    wt_tail = wt[:, MAIN_BLKS * LANE:]
    mesh = plsc.VectorSubcoreMesh(core_axis_name="c", subcore_axis_name="s",
                                  num_cores=NC, num_subcores=NS)
    w128 = pl.kernel(
        _fmt_body,
        out_type=jax.ShapeDtypeStruct((VOCAB, LANE), jnp.float32),
        mesh=mesh,
        scratch_types=[
            pltpu.VMEM((D, LANE), jnp.float32),
            pltpu.VMEM((LANE, LANE), jnp.float32),
            pltpu.VMEM((D, TAIL), jnp.float32),
        ],
        compiler_params=pltpu.CompilerParams(needs_layout_passes=False),
    )(wt_main, wt_tail)
    out = pl.kernel(
        _emb_body,
        out_type=jax.ShapeDtypeStruct((tok_rows, LANE, LANE), jnp.float32),
        mesh=mesh,
        scratch_types=[
            pltpu.VMEM((NB, K, LANE), jnp.int32),
            pltpu.VMEM((NB, K, LANE, LANE), jnp.float32),
            pltpu.SemaphoreType.DMA((NB,)),
        ],
    )(idx, w128)
    return out[:, :, :D].reshape(b0, b1, D)


# speedup vs baseline: 2.4405x; 1.6914x over previous
import jax
import jax.numpy as jnp
from jax import lax
from jax.experimental import pallas as pl
from jax.experimental.pallas import tpu as pltpu
from jax.experimental.pallas import tpu_sc as plsc

D = 64
LANE = 128
NC, NS = 2, 16
NW = NC * NS
NB = 4


def _emb_body(idx_hbm, w_hbm, out_hbm, idx_v, rows_v, gsem):
    tok_rows = idx_hbm.shape[0]
    rpw = tok_rows // NW

    c = lax.axis_index("c")
    s = lax.axis_index("s")
    wid = s * NC + c
    base = wid * rpw

    def start(g, b):
        pltpu.sync_copy(idx_hbm.at[pl.ds(base + g, 1)],
                        idx_v.at[pl.ds(b, 1)])
        pltpu.async_copy(w_hbm.at[idx_v.at[b]], rows_v.at[b], gsem.at[b])

    def finish(g, b):
        pltpu.make_async_copy(w_hbm.at[pl.ds(0, LANE)], rows_v.at[b],
                              gsem.at[b]).wait()
        pltpu.sync_copy(rows_v.at[pl.ds(b, 1)],
                        out_hbm.at[pl.ds(base + g, 1)])

    for b in range(NB):
        start(b, b)

    def loop_body(i, carry):
        g = NB * i
        for b in range(NB):
            finish(g + b, b)
            start(g + NB + b, b)
        return carry

    lax.fori_loop(0, rpw // NB - 1, loop_body, 0)
    for b in range(NB):
        finish(rpw - NB + b, b)


def kernel(tokens, weight):
    b0, b1 = tokens.shape
    vocab = weight.shape[0]
    tok_rows = (b0 * b1) // LANE
    idx = tokens.reshape(tok_rows, LANE)
    w128 = jnp.pad(weight, ((0, 0), (0, LANE - D)))
    mesh = plsc.VectorSubcoreMesh(core_axis_name="c", subcore_axis_name="s",
                                  num_cores=NC, num_subcores=NS)
    out = pl.kernel(
        _emb_body,
        out_type=jax.ShapeDtypeStruct((tok_rows, LANE, LANE), jnp.float32),
        mesh=mesh,
        scratch_types=[
            pltpu.VMEM((NB, LANE), jnp.int32),
            pltpu.VMEM((NB, LANE, LANE), jnp.float32),
            pltpu.SemaphoreType.DMA((NB,)),
        ],
    )(idx, w128)
    return out[:, :, :D].reshape(b0, b1, D)


# speedup vs baseline: 2.4525x; 1.0049x over previous
import jax
import jax.numpy as jnp
from jax import lax
from jax.experimental import pallas as pl
from jax.experimental.pallas import tpu as pltpu
from jax.experimental.pallas import tpu_sc as plsc

D = 64
LANE = 128
NC, NS = 2, 16
NW = NC * NS
NB = 5


def _emb_body(idx_hbm, w_hbm, out_hbm, idx_v, rows_v, gsem, osem):
    tok_rows = idx_hbm.shape[0]
    rpw = tok_rows // NW

    c = lax.axis_index("c")
    s = lax.axis_index("s")
    wid = s * NC + c
    base = wid * rpw

    def start(g, b, first=False):
        pltpu.sync_copy(idx_hbm.at[pl.ds(base + g, 1)],
                        idx_v.at[pl.ds(b, 1)])
        if not first:
            pltpu.make_async_copy(rows_v.at[pl.ds(b, 1)],
                                  out_hbm.at[pl.ds(0, 1)], osem.at[b]).wait()
        pltpu.async_copy(w_hbm.at[idx_v.at[b]], rows_v.at[b], gsem.at[b])

    def finish(g, b):
        pltpu.make_async_copy(w_hbm.at[pl.ds(0, LANE)], rows_v.at[b],
                              gsem.at[b]).wait()
        pltpu.async_copy(rows_v.at[pl.ds(b, 1)],
                         out_hbm.at[pl.ds(base + g, 1)], osem.at[b])

    for b in range(NB):
        start(b, b, first=True)

    def loop_body(i, carry):
        g = NB * i
        for b in range(NB):
            finish(g + b, b)
            start(g + NB + b, b)
        return carry

    lax.fori_loop(0, rpw // NB - 1, loop_body, 0)
    for b in range(NB):
        finish(rpw - NB + b, b)
    for b in range(NB):
        pltpu.make_async_copy(rows_v.at[pl.ds(b, 1)], out_hbm.at[pl.ds(0, 1)],
                              osem.at[b]).wait()


def kernel(tokens, weight):
    b0, b1 = tokens.shape
    vocab = weight.shape[0]
    tok_rows = (b0 * b1) // LANE
    idx = tokens.reshape(tok_rows, LANE)
    w128 = jnp.pad(weight, ((0, 0), (0, LANE - D)))
    mesh = plsc.VectorSubcoreMesh(core_axis_name="c", subcore_axis_name="s",
                                  num_cores=NC, num_subcores=NS)
    out = pl.kernel(
        _emb_body,
        out_type=jax.ShapeDtypeStruct((tok_rows, LANE, LANE), jnp.float32),
        mesh=mesh,
        scratch_types=[
            pltpu.VMEM((NB, LANE), jnp.int32),
            pltpu.VMEM((NB, LANE, LANE), jnp.float32),
            pltpu.SemaphoreType.DMA((NB,)),
            pltpu.SemaphoreType.DMA((NB,)),
        ],
    )(idx, w128)
    return out[:, :, :D].reshape(b0, b1, D)
